# baseline (speedup 1.0000x reference)
import functools

import jax
import jax.numpy as jnp
from jax import lax
from jax.experimental import pallas as pl
from jax.experimental.pallas import tpu as pltpu
from jax.experimental.pallas import tpu_sc as plsc

N = 8192
B = 16
K = 8
H = 16
RT = 512
CT = 512
NI = N // RT
NJ = N // CT
IDX_BIG = 2 ** 30


def _elu(x):
    return jnp.where(x > 0, x, jnp.exp(jnp.where(x > 0, 0.0, x)) - 1.0)


def _prep_body(x_ref, w1_ref, b1_ref, w2_ref, b2_ref, wcb_ref, wd_ref,
               bc_ref, h_ref, m_ref, a_ref):
    x = x_ref[...]
    h = _elu(jax.lax.dot_general(x, w1_ref[...], (((1,), (0,)), ((), ())),
                                 preferred_element_type=jnp.float32)
             + b1_ref[...])
    h = _elu(jax.lax.dot_general(h, w2_ref[...], (((1,), (0,)), ((), ())),
                                 preferred_element_type=jnp.float32)
             + b2_ref[...])
    h_ref[...] = h
    m_ref[...] = jax.lax.dot_general(h, wcb_ref[...], (((1,), (0,)), ((), ())),
                                     preferred_element_type=jnp.float32)
    a_ref[...] = jax.lax.dot_general(h, wd_ref[...], (((1,), (0,)), ((), ())),
                                     preferred_element_type=jnp.float32) + bc_ref[...]


def _derive_body(x_ref, wcb_ref, wd_ref, bc_ref, m_ref, a_ref):
    h = x_ref[...]
    m_ref[...] = jax.lax.dot_general(h, wcb_ref[...], (((1,), (0,)), ((), ())),
                                     preferred_element_type=jnp.float32)
    a_ref[...] = jax.lax.dot_general(h, wd_ref[...], (((1,), (0,)), ((), ())),
                                     preferred_element_type=jnp.float32) + bc_ref[...]


SB = 64
IDX_BIGF = float(2 ** 24)


def _merge_sorted(bv, bi, cvr, cir):
    v = jnp.concatenate([bv, cvr], axis=1)
    x = jnp.concatenate([bi, cir], axis=1)
    for d in (K, K // 2, K // 4, K // 8):
        nv, nx = [], []
        for s in range(0, 2 * K, 2 * d):
            av, bv_ = v[:, s:s + d], v[:, s + d:s + 2 * d]
            ax, bx = x[:, s:s + d], x[:, s + d:s + 2 * d]
            sw = (bv_ < av) | ((bv_ == av) & (bx < ax))
            nv += [jnp.where(sw, bv_, av), jnp.where(sw, av, bv_)]
            nx += [jnp.where(sw, bx, ax), jnp.where(sw, ax, bx)]
        v = jnp.concatenate(nv, axis=1)
        x = jnp.concatenate(nx, axis=1)
    return v[:, :K], x[:, :K]


def _topk_body(valid_ref, fetch_ref, hrow_ref, hcol_ref, brow_ref, bcolT_ref,
               out_ref, bval, bidx):
    i = pl.program_id(0)
    j = pl.program_id(1)

    @pl.when(j == 0)
    def _init():
        bval[...] = jnp.full((RT, K), jnp.inf, jnp.float32)
        bidx[...] = jnp.full((RT, K), IDX_BIGF, jnp.float32)

    step = i * NJ + j
    valid = valid_ref[step]

    @pl.when(valid != 0)
    def _compute():
        hc = hcol_ref[...]
        sqc = jnp.sum(hc * hc, axis=1)
        bcol = bcolT_ref[...]
        coff = (fetch_ref[step] * CT).astype(jnp.float32)
        citf = jax.lax.broadcasted_iota(jnp.int32, (SB, CT), 1).astype(
            jnp.float32)
        for sb in range(RT // SB):
            r0 = sb * SB
            hr = hrow_ref[pl.ds(r0, SB), :]
            sqr = jnp.sum(hr * hr, axis=1)
            dots = jax.lax.dot_general(hr, hc, (((1,), (1,)), ((), ())),
                                       preferred_element_type=jnp.float32)
            d2 = sqr[:, None] + sqc[None, :] - 2.0 * dots
            cross = brow_ref[pl.ds(r0, SB), :] != bcol
            d2 = jnp.where(cross, jnp.inf, d2)
            bv = bval[pl.ds(r0, SB), :]
            imp = jnp.any(d2 < bv[:, K - 1:K])

            @pl.when(imp)
            def _extract(d2=d2, bv=bv, r0=r0):
                bi = bidx[pl.ds(r0, SB), :]
                mvs, mis = [], []
                for _ in range(K):
                    mv = jnp.min(d2, axis=1)
                    im = jnp.where(d2 == mv[:, None], citf, IDX_BIGF)
                    mi = jnp.min(im, axis=1)
                    d2 = jnp.where(im == mi[:, None], jnp.inf, d2)
                    mvs.append(mv[:, None])
                    mis.append(mi[:, None] + coff)
                cvr = jnp.concatenate(mvs[::-1], axis=1)
                cir = jnp.concatenate(mis[::-1], axis=1)
                nbv, nbi = _merge_sorted(bv, bi, cvr, cir)
                bval[pl.ds(r0, SB), :] = nbv
                bidx[pl.ds(r0, SB), :] = nbi

    out_ref[...] = jnp.clip(bidx[...], 0.0, float(N - 1)).astype(jnp.int32)


def _make_topk(interpret=False):
    grid_spec = pltpu.PrefetchScalarGridSpec(
        num_scalar_prefetch=2,
        grid=(NI, NJ),
        in_specs=[
            pl.BlockSpec((RT, H), lambda i, j, v, f: (i, 0)),
            pl.BlockSpec((CT, H), lambda i, j, v, f: (f[i * NJ + j], 0)),
            pl.BlockSpec((RT, 1), lambda i, j, v, f: (i, 0)),
            pl.BlockSpec((1, CT), lambda i, j, v, f: (0, f[i * NJ + j])),
        ],
        out_specs=pl.BlockSpec((RT, K), lambda i, j, v, f: (i, 0)),
        scratch_shapes=[
            pltpu.VMEM((RT, K), jnp.float32),
            pltpu.VMEM((RT, K), jnp.float32),
        ],
    )
    return pl.pallas_call(
        _topk_body,
        grid_spec=grid_spec,
        out_shape=jax.ShapeDtypeStruct((N, K), jnp.int32),
        interpret=interpret,
    )


def _topk_maps(batch):
    bs = batch[::RT]
    be = batch[RT - 1::RT]
    valid = (bs[None, :] <= be[:, None]) & (be[None, :] >= bs[:, None])
    jlo = jnp.argmax(valid, axis=1).astype(jnp.int32)
    jhi = (NJ - 1) - jnp.argmax(valid[:, ::-1], axis=1).astype(jnp.int32)
    fetch = jnp.clip(jnp.arange(NJ, dtype=jnp.int32)[None, :],
                     jlo[:, None], jhi[:, None])
    return valid.astype(jnp.int32).reshape(-1), fetch.reshape(-1)


_NW = 32
_NPW = N // _NW
_CH = 128
_NCHUNK = _NPW * K // _CH
_NPC = _CH // K
_MW = 128


def _sc_gather_body(m_hbm, a_hbm, idx_hbm, out_hbm, idx_v, rows0, rows1, a_v,
                    f_v, sem):
    wid = lax.axis_index("s") * 2 + lax.axis_index("c")
    base = wid * _NPW
    pltpu.sync_copy(idx_hbm.at[pl.ds(base * K, _NPW * K)], idx_v)
    pltpu.sync_copy(a_hbm.at[pl.ds(base * H, _NPW * H)], a_v)
    bufs = (rows0, rows1)

    def fire(c):
        return pltpu.async_copy(
            m_hbm.at[idx_v.at[pl.ds(c * _CH, _CH)]], bufs[c % 2], sem)

    pending = fire(0)
    for c in range(_NCHUNK):
        pending.wait()
        if c + 1 < _NCHUNK:
            pending = fire(c + 1)
        buf = bufs[c % 2]
        for nl in range(_NPC):
            n = c * _NPC + nl
            av = a_v[pl.ds(n * H, H)]
            msg = av + buf[nl * K, pl.ds(0, H)]
            acc = jnp.where(msg > 0, msg, jnp.exp(msg) - 1.0)
            for kk in range(1, K):
                msg = av + buf[nl * K + kk, pl.ds(0, H)]
                acc = jnp.maximum(acc,
                                  jnp.where(msg > 0, msg, jnp.exp(msg) - 1.0))
            f_v[pl.ds(n * H, H)] = acc
    pltpu.sync_copy(f_v, out_hbm.at[pl.ds(base * H, _NPW * H)])


def _sc_gather(m_pad, a_flat, idx_flat):
    mesh = plsc.VectorSubcoreMesh(core_axis_name="c", subcore_axis_name="s")
    fn = functools.partial(
        pl.kernel,
        out_type=jax.ShapeDtypeStruct((N * H,), jnp.float32),
        mesh=mesh,
        scratch_types=[
            pltpu.VMEM((_NPW * K,), jnp.int32),
            pltpu.VMEM((_CH, _MW), jnp.float32),
            pltpu.VMEM((_CH, _MW), jnp.float32),
            pltpu.VMEM((_NPW * H,), jnp.float32),
            pltpu.VMEM((_NPW * H,), jnp.float32),
            pltpu.SemaphoreType.DMA,
        ],
    )(_sc_gather_body)
    return fn(m_pad, a_flat, idx_flat).reshape(N, H)


def _final_body(f2_ref, bT_ref, wo1_ref, bo1_ref, wo2_ref, bo2_ref, wo3_ref,
                bo3_ref, out_ref):
    f2 = f2_ref[...]
    bT = bT_ref[...]
    rows = jax.lax.broadcasted_iota(jnp.int32, (B, N), 0)
    oh = (rows == bT).astype(jnp.float32)
    cnt = jnp.sum(oh, axis=1)
    s = jax.lax.dot_general(oh, f2, (((1,), (0,)), ((), ())),
                            preferred_element_type=jnp.float32)
    pooled = s / jnp.maximum(cnt, 1.0)[:, None]
    o = _elu(jax.lax.dot_general(pooled, wo1_ref[...], (((1,), (0,)), ((), ())),
                                 preferred_element_type=jnp.float32)
             + bo1_ref[...])
    o = _elu(jax.lax.dot_general(o, wo2_ref[...], (((1,), (0,)), ((), ())),
                                 preferred_element_type=jnp.float32)
             + bo2_ref[...])
    o = jax.lax.dot_general(o, wo3_ref[...], (((1,), (0,)), ((), ())),
                            preferred_element_type=jnp.float32) + bo3_ref[...]
    out_ref[...] = o


@jax.jit
def _run(x_pf, batch_pf, W1, b1, W2, b2, Wc, bc, Wo1, bo1, Wo2, bo2, Wo3, bo3):
    batch = batch_pf.astype(jnp.int32)
    b2d = batch.reshape(N, 1)
    bT = batch.reshape(1, N)
    wcb = jnp.pad(Wc[H:], ((0, 0), (0, _MW - H)))
    wd = Wc[:H] - Wc[H:]

    h, m1, a1 = pl.pallas_call(
        _prep_body,
        out_shape=[jax.ShapeDtypeStruct((N, H), jnp.float32),
                   jax.ShapeDtypeStruct((N, _MW), jnp.float32),
                   jax.ShapeDtypeStruct((N, H), jnp.float32)],
    )(x_pf, W1, b1.reshape(1, -1), W2, b2.reshape(1, -1), wcb, wd,
      bc.reshape(1, -1))

    valid, fetch = _topk_maps(batch)
    topk = _make_topk()
    idx1 = topk(valid, fetch, h, h, b2d, bT)
    f1 = _sc_gather(m1, a1.reshape(-1), idx1.reshape(-1))

    m2, a2 = pl.pallas_call(
        _derive_body,
        out_shape=[jax.ShapeDtypeStruct((N, _MW), jnp.float32),
                   jax.ShapeDtypeStruct((N, H), jnp.float32)],
    )(f1, wcb, wd, bc.reshape(1, -1))
    idx2 = topk(valid, fetch, f1, f1, b2d, bT)
    f2 = _sc_gather(m2, a2.reshape(-1), idx2.reshape(-1))

    o = pl.pallas_call(
        _final_body,
        out_shape=jax.ShapeDtypeStruct((B, 1), jnp.float32),
    )(f2, bT, Wo1, bo1.reshape(1, -1), Wo2, bo2.reshape(1, -1), Wo3,
      bo3.reshape(1, -1))
    return o, jnp.arange(B, dtype=jnp.int32)


def kernel(x_pf, batch_pf, W1, b1, W2, b2, Wc, bc, Wo1, bo1, Wo2, bo2, Wo3,
           bo3):
    return _run(x_pf, batch_pf, W1, b1, W2, b2, Wc, bc, Wo1, bo1, Wo2, bo2,
                Wo3, bo3)


# speedup vs baseline: 2.6940x; 2.6940x over previous
import functools

import jax
import jax.numpy as jnp
from jax import lax
from jax.experimental import pallas as pl
from jax.experimental.pallas import tpu as pltpu
from jax.experimental.pallas import tpu_sc as plsc

N = 8192
B = 16
K = 8
H = 16
MW = 128
RT = 512
CT = 512
NI = N // RT
NJ = N // CT
IDX_BIGF = float(2 ** 24)


def _lin_body(x_ref, w_ref, b_ref, o_ref):
    o_ref[...] = jax.lax.dot_general(
        x_ref[...], w_ref[...], (((1,), (0,)), ((), ())),
        preferred_element_type=jnp.float32) + b_ref[...]


def _linear(x, w, b, out_w):
    return pl.pallas_call(
        _lin_body,
        out_shape=jax.ShapeDtypeStruct((x.shape[0], out_w), jnp.float32),
    )(x, w, b)


def _merge_sorted(bv, bi, cvr, cir):
    v = jnp.concatenate([bv, cvr], axis=0)
    x = jnp.concatenate([bi, cir], axis=0)
    for d in (K, K // 2, K // 4, K // 8):
        nv, nx = [], []
        for s in range(0, 2 * K, 2 * d):
            av, bv_ = v[s:s + d], v[s + d:s + 2 * d]
            ax, bx = x[s:s + d], x[s + d:s + 2 * d]
            sw = (bv_ < av) | ((bv_ == av) & (bx < ax))
            nv += [jnp.where(sw, bv_, av), jnp.where(sw, av, bv_)]
            nx += [jnp.where(sw, bx, ax), jnp.where(sw, ax, bx)]
        v = jnp.concatenate(nv, axis=0)
        x = jnp.concatenate(nx, axis=0)
    return v[:K], x[:K]


def _topk_body(valid_ref, fetch_ref, hrow_ref, hcol_ref, bcol_ref, browT_ref,
               out_ref, bval, bidx):
    i = pl.program_id(0)
    j = pl.program_id(1)

    @pl.when(j == 0)
    def _init():
        bval[...] = jnp.full((K, RT), jnp.inf, jnp.float32)
        bidx[...] = jnp.full((K, RT), IDX_BIGF, jnp.float32)

    step = i * NJ + j
    valid = valid_ref[step]

    @pl.when(valid != 0)
    def _compute():
        hr = hrow_ref[...]
        hc = hcol_ref[...]
        sqr = jnp.sum(hr * hr, axis=1)
        sqc = jnp.sum(hc * hc, axis=1)
        dots = jax.lax.dot_general(hc, hr, (((1,), (1,)), ((), ())),
                                   preferred_element_type=jnp.float32)
        d2 = sqc[:, None] + sqr[None, :] - 2.0 * dots
        cross = bcol_ref[...] != browT_ref[...]
        d2 = jnp.where(cross, jnp.inf, d2)
        imp = jnp.any(d2 < bval[K - 1:K, :])

        @pl.when(imp)
        def _extract():
            citf = jax.lax.broadcasted_iota(jnp.int32, (CT, RT), 0).astype(
                jnp.float32)
            coff = (fetch_ref[step] * CT).astype(jnp.float32)
            d2l = d2
            mvs, mis = [], []
            for _ in range(K):
                mv = jnp.min(d2l, axis=0)
                im = jnp.where(d2l == mv[None, :], citf, IDX_BIGF)
                mi = jnp.min(im, axis=0)
                d2l = jnp.where(im == mi[None, :], jnp.inf, d2l)
                mvs.append(mv[None, :])
                mis.append(mi[None, :] + coff)
            cvr = jnp.concatenate(mvs[::-1], axis=0)
            cir = jnp.concatenate(mis[::-1], axis=0)
            nbv, nbi = _merge_sorted(bval[...], bidx[...], cvr, cir)
            bval[...] = nbv
            bidx[...] = nbi

    out_ref[...] = jnp.clip(bidx[...], 0.0, float(N - 1)).astype(
        jnp.int32)[None]


def _make_topk(interpret=False):
    grid_spec = pltpu.PrefetchScalarGridSpec(
        num_scalar_prefetch=2,
        grid=(NI, NJ),
        in_specs=[
            pl.BlockSpec((RT, MW), lambda i, j, v, f: (i, 0)),
            pl.BlockSpec((CT, MW), lambda i, j, v, f: (f[i * NJ + j], 0)),
            pl.BlockSpec((CT, 1), lambda i, j, v, f: (f[i * NJ + j], 0)),
            pl.BlockSpec((1, RT), lambda i, j, v, f: (0, i)),
        ],
        out_specs=pl.BlockSpec((1, K, RT), lambda i, j, v, f: (i, 0, 0)),
        scratch_shapes=[
            pltpu.VMEM((K, RT), jnp.float32),
            pltpu.VMEM((K, RT), jnp.float32),
        ],
    )
    return pl.pallas_call(
        _topk_body,
        grid_spec=grid_spec,
        out_shape=jax.ShapeDtypeStruct((NI, K, RT), jnp.int32),
        interpret=interpret,
    )


def _topk_call(valid, fetch, xp, b2d, bT, interpret=False):
    raw = _make_topk(interpret)(valid, fetch, xp, xp, b2d, bT)
    return jnp.transpose(raw, (0, 2, 1)).reshape(N, K)


def _topk_maps(batch):
    bs = batch[::RT]
    be = batch[RT - 1::RT]
    valid = (bs[None, :] <= be[:, None]) & (be[None, :] >= bs[:, None])
    jlo = jnp.argmax(valid, axis=1).astype(jnp.int32)
    jhi = (NJ - 1) - jnp.argmax(valid[:, ::-1], axis=1).astype(jnp.int32)
    fetch = jnp.clip(jnp.arange(NJ, dtype=jnp.int32)[None, :],
                     jlo[:, None], jhi[:, None])
    return valid.astype(jnp.int32).reshape(-1), fetch.reshape(-1)


_NW = 32
_NPW = N // _NW
_CH = 128
_NCHUNK = _NPW * K // _CH
_NPC = _CH // K


def _sc_gather_body(t_hbm, idx_hbm, out_hbm, idx_v, rows0, rows1, pk_v, sem):
    wid = lax.axis_index("s") * 2 + lax.axis_index("c")
    base = wid * _NPW
    pltpu.sync_copy(idx_hbm.at[pl.ds(base * K, _NPW * K)], idx_v)
    bufs = (rows0, rows1)

    def fire(c):
        return pltpu.async_copy(
            t_hbm.at[idx_v.at[pl.ds(c * _CH, _CH)]], bufs[c % 2], sem)

    pending = fire(0)
    for c in range(_NCHUNK):
        pending.wait()
        if c + 1 < _NCHUNK:
            pending = fire(c + 1)
        buf = bufs[c % 2]
        for m in range(_NPC):
            n = c * _NPC + m
            for kk in range(K):
                pk_v[n, pl.ds(kk * H, H)] = buf[m * K + kk, pl.ds(0, H)]
    pltpu.sync_copy(pk_v, out_hbm.at[pl.ds(base, _NPW), :])


def _sc_gather(table_p, idx_flat):
    mesh = plsc.VectorSubcoreMesh(core_axis_name="c", subcore_axis_name="s")
    fn = functools.partial(
        pl.kernel,
        out_type=jax.ShapeDtypeStruct((N, MW), jnp.float32),
        mesh=mesh,
        scratch_types=[
            pltpu.VMEM((_NPW * K,), jnp.int32),
            pltpu.VMEM((_CH, MW), jnp.float32),
            pltpu.VMEM((_CH, MW), jnp.float32),
            pltpu.VMEM((_NPW, MW), jnp.float32),
            pltpu.SemaphoreType.DMA,
        ],
    )(_sc_gather_body)
    return fn(table_p, idx_flat)


def _msg_body(xp_ref, xjp_ref, wc_ref, bc_ref, out_ref):
    xi = xp_ref[:, :H]
    xjp = xjp_ref[...]
    xi_rep = jnp.concatenate([xi] * K, axis=0)
    xj_km = jnp.concatenate([xjp[:, kk * H:(kk + 1) * H] for kk in range(K)],
                            axis=0)
    cat = jnp.concatenate([xi_rep, xj_km - xi_rep], axis=1)
    pre = jax.lax.dot_general(cat, wc_ref[...], (((1,), (0,)), ((), ())),
                              preferred_element_type=jnp.float32)
    m = pre
    size = RT * K
    while size > RT:
        size //= 2
        m = jnp.maximum(m[:size], m[size:])
    m = m + bc_ref[...]
    out_ref[...] = jnp.concatenate(
        [m, jnp.zeros((RT, MW - H), jnp.float32)], axis=1)


def _msg_call(xp, xjp, Wc, bc, interpret=False):
    return pl.pallas_call(
        _msg_body,
        grid=(NI,),
        in_specs=[
            pl.BlockSpec((RT, MW), lambda i: (i, 0)),
            pl.BlockSpec((RT, MW), lambda i: (i, 0)),
            pl.BlockSpec((2 * H, H), lambda i: (0, 0)),
            pl.BlockSpec((1, H), lambda i: (0, 0)),
        ],
        out_specs=pl.BlockSpec((RT, MW), lambda i: (i, 0)),
        out_shape=jax.ShapeDtypeStruct((N, MW), jnp.float32),
        interpret=interpret,
    )(xp, xjp, Wc, bc)


def _final_body(f2_ref, bT_ref, wo1_ref, bo1_ref, wo2_ref, bo2_ref, wo3_ref,
                bo3_ref, out_ref):
    f2 = f2_ref[:, :H]
    bT = bT_ref[...]
    rows = jax.lax.broadcasted_iota(jnp.int32, (B, N), 0)
    oh = (rows == bT).astype(jnp.float32)
    cnt = jnp.sum(oh, axis=1)
    s = jax.lax.dot_general(oh, f2, (((1,), (0,)), ((), ())),
                            preferred_element_type=jnp.float32)
    pooled = s / jnp.maximum(cnt, 1.0)[:, None]

    def _elu(x):
        return jnp.where(x > 0, x, jnp.exp(jnp.where(x > 0, 0.0, x)) - 1.0)

    o = _elu(jax.lax.dot_general(pooled, wo1_ref[...], (((1,), (0,)), ((), ())),
                                 preferred_element_type=jnp.float32)
             + bo1_ref[...])
    o = _elu(jax.lax.dot_general(o, wo2_ref[...], (((1,), (0,)), ((), ())),
                                 preferred_element_type=jnp.float32)
             + bo2_ref[...])
    o = jax.lax.dot_general(o, wo3_ref[...], (((1,), (0,)), ((), ())),
                            preferred_element_type=jnp.float32) + bo3_ref[...]
    out_ref[...] = o


@jax.jit
def _run(x_pf, batch_pf, W1, b1, W2, b2, Wc, bc, Wo1, bo1, Wo2, bo2, Wo3, bo3):
    batch = batch_pf.astype(jnp.int32)
    b2d = batch.reshape(N, 1)
    bT = batch.reshape(1, N)
    w2p = jnp.pad(W2, ((0, 0), (0, MW - H)))
    b2p = jnp.pad(b2.reshape(1, -1), ((0, 0), (0, MW - H)))

    l1 = _linear(x_pf, W1, b1.reshape(1, -1), H)
    hp = jax.nn.elu(_linear(jax.nn.elu(l1), w2p, b2p, MW))

    valid, fetch = _topk_maps(batch)
    bcr = bc.reshape(1, -1)

    def edge_conv(xp):
        idx = _topk_call(valid, fetch, xp, b2d, bT)
        xjp = _sc_gather(xp, idx.reshape(-1))
        return jax.nn.elu(_msg_call(xp, xjp, Wc, bcr))

    f1p = edge_conv(hp)
    f2p = edge_conv(f1p)

    o = pl.pallas_call(
        _final_body,
        out_shape=jax.ShapeDtypeStruct((B, 1), jnp.float32),
    )(f2p, bT, Wo1, bo1.reshape(1, -1), Wo2, bo2.reshape(1, -1), Wo3,
      bo3.reshape(1, -1))
    return o, jnp.arange(B, dtype=jnp.int32)


def kernel(x_pf, batch_pf, W1, b1, W2, b2, Wc, bc, Wo1, bo1, Wo2, bo2, Wo3,
           bo3):
    return _run(x_pf, batch_pf, W1, b1, W2, b2, Wc, bc, Wo1, bo1, Wo2, bo2,
                Wo3, bo3)
